# baseline (speedup 1.0000x reference)
import functools

import jax
import jax.numpy as jnp
from jax import lax
from jax.experimental import pallas as pl
from jax.experimental.pallas import tpu as pltpu
from jax.experimental.pallas import tpu_sc as plsc

B = 128
V = 100000
VSC = 57600
VT = V - VSC
TCB = 800
NC = 2
NS = 16
NW = NC * NS
S = 1888
VC = 472
NCHK = S // VC
BIG = 2**31 - 1
NEG_INF = float("-inf")

_mesh = plsc.VectorSubcoreMesh(core_axis_name="c", subcore_axis_name="s")


@functools.partial(
    pl.kernel,
    out_type=[jax.ShapeDtypeStruct((NW * B,), jnp.int32),
              jax.ShapeDtypeStruct((NW * B,), jnp.float32)],
    mesh=_mesh,
    scratch_types=[
        pltpu.VMEM((2, VC, B), jnp.float32),
        pltpu.VMEM((16,), jnp.float32),
        pltpu.VMEM((16,), jnp.int32),
        pltpu.SemaphoreType.DMA,
        pltpu.SemaphoreType.DMA,
    ],
)
def _sc_scan(xt_hbm, outi_hbm, outv_hbm, buf, sv, si, sem0, sem1):
    cid = lax.axis_index("c")
    sid = lax.axis_index("s")
    wid = cid * NS + sid
    v0 = pl.multiple_of((wid * (VSC - S) // (NW - 1)) // 8 * 8, 8)
    sems = (sem0, sem1)
    lanes = lax.iota(jnp.int32, 16)
    zero_i = lanes * 0
    neginf_f = zero_i.astype(jnp.float32) + NEG_INF

    def start(k):
        return pltpu.async_copy(
            xt_hbm.at[pl.ds(v0 + k * VC, VC), :], buf.at[k % 2],
            sems[k % 2])

    bvs = [neginf_f for _ in range(8)]
    bis = [zero_i for _ in range(8)]

    descs = [None, None]
    descs[0] = start(0)
    for k in range(NCHK):
        if k + 1 < NCHK:
            descs[(k + 1) % 2] = start(k + 1)
        descs[k % 2].wait()
        bref = buf.at[k % 2]
        cbase = v0 + k * VC

        def it(v, carry):
            accs, civ = carry
            out = []
            for lg in range(8):
                bv, bi = accs[lg]
                x = bref[v, pl.ds(lg * 16, 16)]
                gt = x > bv
                bv = jnp.maximum(bv, x)
                bi = jnp.where(gt, civ, bi)
                out.append((bv, bi))
            return tuple(out), civ + 1

        civ0 = zero_i + cbase
        accs, _ = lax.fori_loop(
            0, VC, it,
            (tuple((bvs[lg], bis[lg]) for lg in range(8)), civ0),
            unroll=8)
        for lg in range(8):
            bvs[lg], bis[lg] = accs[lg]

    for lg in range(8):
        si[...] = bis[lg]
        pltpu.sync_copy(si, outi_hbm.at[pl.ds(wid * B + lg * 16, 16)])
        sv[...] = bvs[lg]
        pltpu.sync_copy(sv, outv_hbm.at[pl.ds(wid * B + lg * 16, 16)])


def _tc_body(x_ref, ov_ref, oi_ref):
    i = pl.program_id(0)
    x = x_ref[...]
    rows = (jax.lax.broadcasted_iota(jnp.int32, (TCB, B), 0)
            + i * TCB + VSC)
    m = jnp.max(x, axis=0, keepdims=True)
    idx = jnp.min(jnp.where(x == m, rows, BIG), axis=0, keepdims=True)

    @pl.when(i == 0)
    def _():
        ov_ref[...] = m
        oi_ref[...] = idx

    @pl.when(i > 0)
    def _():
        pv = ov_ref[...]
        pi = oi_ref[...]
        gt = m > pv
        eq = m == pv
        oi_ref[...] = jnp.where(gt, idx,
                                jnp.where(eq, jnp.minimum(idx, pi), pi))
        ov_ref[...] = jnp.maximum(m, pv)


_tc_scan = pl.pallas_call(
    _tc_body,
    grid=(VT // TCB,),
    in_specs=[pl.BlockSpec((TCB, B), lambda i: (VSC // TCB + i, 0))],
    out_specs=[pl.BlockSpec((1, B), lambda i: (0, 0)),
               pl.BlockSpec((1, B), lambda i: (0, 0))],
    out_shape=[jax.ShapeDtypeStruct((1, B), jnp.float32),
               jax.ShapeDtypeStruct((1, B), jnp.int32)],
)


@functools.partial(
    pl.kernel,
    out_type=jax.ShapeDtypeStruct((B,), jnp.int32),
    mesh=_mesh,
    scratch_types=[
        pltpu.VMEM((NW * B,), jnp.float32),
        pltpu.VMEM((NW * B,), jnp.int32),
        pltpu.VMEM((B,), jnp.float32),
        pltpu.VMEM((B,), jnp.int32),
        pltpu.VMEM((16,), jnp.int32),
        pltpu.SemaphoreType.DMA,
        pltpu.SemaphoreType.DMA,
        pltpu.SemaphoreType.DMA,
        pltpu.SemaphoreType.DMA,
    ],
)
def _sc_merge(pi_hbm, pv_hbm, ti_hbm, tv_hbm, out_hbm,
              vbuf, ibuf, tvbuf, tibuf, si, sem0, sem1, sem2, sem3):
    cid = lax.axis_index("c")
    sid = lax.axis_index("s")
    lanes = lax.iota(jnp.int32, 16)
    zero_i = lanes * 0
    neginf_f = zero_i.astype(jnp.float32) + NEG_INF

    @pl.when(sid % 4 == 0)
    def _():
        a = cid * 4 + sid // 4
        b0 = a * 16
        d0 = pltpu.async_copy(pv_hbm, vbuf, sem0)
        d1 = pltpu.async_copy(pi_hbm, ibuf, sem1)
        d2 = pltpu.async_copy(tv_hbm, tvbuf, sem2)
        d3 = pltpu.async_copy(ti_hbm, tibuf, sem3)
        d0.wait()
        d1.wait()
        d2.wait()
        d3.wait()
        bv = neginf_f
        bi = zero_i

        def fold(v, i, bv, bi):
            gt = v > bv
            eq = v == bv
            bv2 = jnp.maximum(bv, v)
            bi2 = jnp.where(gt, i, bi)
            bi2 = jnp.where(eq, jnp.minimum(bi2, i), bi2)
            return bv2, bi2

        for w in range(NW):
            bv, bi = fold(vbuf[pl.ds(w * B + b0, 16)],
                          ibuf[pl.ds(w * B + b0, 16)], bv, bi)
        bv, bi = fold(tvbuf[pl.ds(b0, 16)], tibuf[pl.ds(b0, 16)], bv, bi)
        si[...] = bi
        pltpu.sync_copy(si, out_hbm.at[pl.ds(b0, 16)])


def kernel(m_logits):
    xt = m_logits.T
    pi, pv = _sc_scan(xt)
    tv, ti = _tc_scan(xt)
    out = _sc_merge(pi, pv, ti.reshape(B), tv.reshape(B))
    return out.reshape(B, 1).astype(jnp.int64)


# speedup vs baseline: 1.1209x; 1.1209x over previous
import functools

import jax
import jax.numpy as jnp
from jax import lax
from jax.experimental import pallas as pl
from jax.experimental.pallas import tpu as pltpu
from jax.experimental.pallas import tpu_sc as plsc

B = 128
V = 100000
NC = 2
NS = 16
NW = NC * NS
S = 3136
VC = 448
NCHK = S // VC
NEG_INF = float("-inf")

_mesh = plsc.VectorSubcoreMesh(core_axis_name="c", subcore_axis_name="s")


@functools.partial(
    pl.kernel,
    out_type=[jax.ShapeDtypeStruct((NW * B,), jnp.int32),
              jax.ShapeDtypeStruct((NW * B,), jnp.float32)],
    mesh=_mesh,
    scratch_types=[
        pltpu.VMEM((2, VC, B), jnp.float32),
        pltpu.VMEM((16,), jnp.float32),
        pltpu.VMEM((16,), jnp.int32),
        pltpu.SemaphoreType.DMA,
        pltpu.SemaphoreType.DMA,
    ],
)
def _sc_scan(xt_hbm, outi_hbm, outv_hbm, buf, sv, si, sem0, sem1):
    cid = lax.axis_index("c")
    sid = lax.axis_index("s")
    wid = cid * NS + sid
    v0 = pl.multiple_of((wid * (V - S) // (NW - 1)) // 8 * 8, 8)
    sems = (sem0, sem1)
    lanes = lax.iota(jnp.int32, 16)
    zero_i = lanes * 0
    neginf_f = zero_i.astype(jnp.float32) + NEG_INF

    def start(k):
        return pltpu.async_copy(
            xt_hbm.at[pl.ds(v0 + k * VC, VC), :], buf.at[k % 2],
            sems[k % 2])

    bvs = [neginf_f for _ in range(8)]
    bis = [zero_i for _ in range(8)]

    descs = [None, None]
    descs[0] = start(0)
    for k in range(NCHK):
        if k + 1 < NCHK:
            descs[(k + 1) % 2] = start(k + 1)
        descs[k % 2].wait()
        bref = buf.at[k % 2]
        cbase = v0 + k * VC

        def it(v, carry):
            accs, civ = carry
            out = []
            for lg in range(8):
                bv, bi = accs[lg]
                x = bref[v, pl.ds(lg * 16, 16)]
                gt = x > bv
                bv = jnp.maximum(bv, x)
                bi = jnp.where(gt, civ, bi)
                out.append((bv, bi))
            return tuple(out), civ + 1

        civ0 = zero_i + cbase
        accs, _ = lax.fori_loop(
            0, VC, it,
            (tuple((bvs[lg], bis[lg]) for lg in range(8)), civ0),
            unroll=8)
        for lg in range(8):
            bvs[lg], bis[lg] = accs[lg]

    for lg in range(8):
        si[...] = bis[lg]
        pltpu.sync_copy(si, outi_hbm.at[pl.ds(wid * B + lg * 16, 16)])
        sv[...] = bvs[lg]
        pltpu.sync_copy(sv, outv_hbm.at[pl.ds(wid * B + lg * 16, 16)])


@functools.partial(
    pl.kernel,
    out_type=jax.ShapeDtypeStruct((B,), jnp.int32),
    mesh=_mesh,
    scratch_types=[
        pltpu.VMEM((NW * B,), jnp.float32),
        pltpu.VMEM((NW * B,), jnp.int32),
        pltpu.VMEM((16,), jnp.int32),
        pltpu.SemaphoreType.DMA,
        pltpu.SemaphoreType.DMA,
    ],
)
def _sc_merge(pi_hbm, pv_hbm, out_hbm, vbuf, ibuf, si, sem0, sem1):
    cid = lax.axis_index("c")
    sid = lax.axis_index("s")
    lanes = lax.iota(jnp.int32, 16)
    zero_i = lanes * 0
    neginf_f = zero_i.astype(jnp.float32) + NEG_INF

    @pl.when(sid % 4 == 0)
    def _():
        a = cid * 4 + sid // 4
        b0 = a * 16
        d0 = pltpu.async_copy(pv_hbm, vbuf, sem0)
        d1 = pltpu.async_copy(pi_hbm, ibuf, sem1)
        d0.wait()
        d1.wait()
        bv = neginf_f
        bi = zero_i
        for w in range(NW):
            v = vbuf[pl.ds(w * B + b0, 16)]
            i = ibuf[pl.ds(w * B + b0, 16)]
            gt = v > bv
            eq = v == bv
            bv = jnp.maximum(bv, v)
            bi = jnp.where(gt, i, bi)
            bi = jnp.where(eq, jnp.minimum(bi, i), bi)
        si[...] = bi
        pltpu.sync_copy(si, out_hbm.at[pl.ds(b0, 16)])


def kernel(m_logits):
    xt = m_logits.T
    pi, pv = _sc_scan(xt)
    out = _sc_merge(pi, pv)
    return out.reshape(B, 1).astype(jnp.int64)
